# baseline (speedup 1.0000x reference)
import jax
import jax.numpy as jnp
from jax import lax
from jax.experimental import pallas as pl
from jax.experimental.pallas import tpu as pltpu
from jax.experimental.pallas import tpu_sc as plsc

BATCH = 16384
EMB_DIM = 64
NC = 2
NS = 16
NW = NC * NS
B_PER_W = BATCH // NW
LANES = 16


def _lookup_body(uid_idx_hbm, iid_idx_hbm, uid_table_hbm, iid_table_hbm,
                 uid_out_hbm, iid_out_hbm,
                 idx_u, idx_i, rows_u, rows_i, sem_u, sem_i, sem_o):
    wid = lax.axis_index("s") * NC + lax.axis_index("c")
    base = wid * B_PER_W
    pltpu.sync_copy(uid_idx_hbm.at[pl.ds(base, B_PER_W)], idx_u)
    pltpu.sync_copy(iid_idx_hbm.at[pl.ds(base, B_PER_W)], idx_i)
    HALF = B_PER_W // 2
    prev = []
    for h in range(2):
        def fetch(g, carry, h=h):
            vec_u = idx_u[pl.ds(h * HALF + g * LANES, LANES)]
            vec_i = idx_i[pl.ds(h * HALF + g * LANES, LANES)]
            for l in range(LANES):
                pltpu.async_copy(uid_table_hbm.at[pl.ds(vec_u[l], 1)],
                                 rows_u.at[pl.ds(g * LANES + l, 1)], sem_u)
                pltpu.async_copy(iid_table_hbm.at[pl.ds(vec_i[l], 1)],
                                 rows_i.at[pl.ds(g * LANES + l, 1)], sem_i)
            return carry

        for c in prev:
            c.wait()
        prev = []
        lax.fori_loop(0, HALF // LANES, fetch, 0)
        pltpu.make_async_copy(
            uid_table_hbm.at[pl.ds(0, HALF)], rows_u, sem_u).wait()
        prev.append(pltpu.async_copy(
            rows_u, uid_out_hbm.at[pl.ds(base + h * HALF, HALF)], sem_o))
        pltpu.make_async_copy(
            iid_table_hbm.at[pl.ds(0, HALF)], rows_i, sem_i).wait()
        prev.append(pltpu.async_copy(
            rows_i, iid_out_hbm.at[pl.ds(base + h * HALF, HALF)], sem_o))
    for c in prev:
        c.wait()


def kernel(x, uid_table, iid_table):
    uid_idx = x[:, 0]
    iid_idx = x[:, 1]
    mesh = plsc.VectorSubcoreMesh(core_axis_name="c", subcore_axis_name="s")
    f = pl.kernel(
        _lookup_body,
        out_type=(
            jax.ShapeDtypeStruct((BATCH, EMB_DIM), jnp.float32),
            jax.ShapeDtypeStruct((BATCH, EMB_DIM), jnp.float32),
        ),
        mesh=mesh,
        scratch_types=[
            pltpu.VMEM((B_PER_W,), jnp.int32),
            pltpu.VMEM((B_PER_W,), jnp.int32),
            pltpu.VMEM((B_PER_W // 2, EMB_DIM), jnp.float32),
            pltpu.VMEM((B_PER_W // 2, EMB_DIM), jnp.float32),
            pltpu.SemaphoreType.DMA,
            pltpu.SemaphoreType.DMA,
            pltpu.SemaphoreType.DMA,
        ],
        compiler_params=pltpu.CompilerParams(needs_layout_passes=False),
    )
    return f(uid_idx, iid_idx, uid_table, iid_table)
